# baseline (speedup 1.0000x reference)
import jax
import jax.numpy as jnp
from jax import lax
from jax.experimental import pallas as pl
from jax.experimental.pallas import tpu as pltpu
from jax.experimental.pallas import tpu_sc as plsc

NUM_RELATION_TYPES = 3
EMBED_SIZE = 300
BATCH = 16384
GROUP = 4

NC = 2
NS = 16
NW = NC * NS
L = 16
L2 = 32
BPW = BATCH // NW
CHUNK = 128
NCHUNK = BPW // CHUNK
GPC = CHUNK // L
DPAD = 320
WROW = DPAD // 2
NSL = WROW // L
CROWS = CHUNK * WROW


def _body(problems_hbm, rels_hbm, targets_hbm, prob_hbm, ord_hbm, rel_hbm,
          out_hbm, pidx_v, tidx_v, ridx_v, rel_v, p0, t0, p1, t1,
          partial_v, sem):
    wid = lax.axis_index("s") * NC + lax.axis_index("c")
    base = wid * BPW

    pltpu.sync_copy(problems_hbm.at[pl.ds(base, BPW)], pidx_v)
    pltpu.sync_copy(targets_hbm.at[pl.ds(base, BPW)], tidx_v)
    pltpu.sync_copy(rels_hbm.at[pl.ds(base, BPW)], ridx_v)
    pltpu.sync_copy(rel_hbm, rel_v)

    iota = lax.iota(jnp.int32, L)

    def issue_group(c, g, p_buf, t_buf):
        goff = pl.multiple_of(g * L, L)
        off = pl.multiple_of(c * CHUNK, CHUNK) + goff
        pidx = pidx_v[pl.ds(off, L)]
        tidx = tidx_v[pl.ds(off, L)]
        for k in range(L):
            dst = pl.multiple_of((goff + k) * WROW, WROW)
            psrc = pl.multiple_of(pidx[k] * WROW, WROW)
            tsrc = pl.multiple_of(tidx[k] * WROW, WROW)
            pltpu.async_copy(prob_hbm.at[pl.ds(psrc, WROW)],
                             p_buf.at[pl.ds(dst, WROW)], sem)
            pltpu.async_copy(ord_hbm.at[pl.ds(tsrc, WROW)],
                             t_buf.at[pl.ds(dst, WROW)], sem)

    def drain():
        pltpu.make_async_copy(prob_hbm.at[pl.ds(0, CROWS)], p0, sem).wait()
        pltpu.make_async_copy(ord_hbm.at[pl.ds(0, CROWS)], t0, sem).wait()

    def compute_group(c, g, p_buf, t_buf, lsum):
        goff = pl.multiple_of(g * L, L)
        coff = pl.multiple_of(c * CHUNK, CHUNK)
        rid_vec = ridx_v[pl.ds(coff + goff, L)]
        cur = jnp.float32(0.0)
        for k in range(L):
            roff = pl.multiple_of((goff + k) * WROW, WROW)
            reloff = pl.multiple_of(rid_vec[k] * WROW, WROW)
            acc = jnp.zeros((L,), jnp.float32)
            for j in range(NSL):
                pw = p_buf[pl.ds(roff + j * L, L)]
                tw = t_buf[pl.ds(roff + j * L, L)]
                rw = rel_v[pl.ds(reloff + j * L, L)]
                prod = (plsc.bitcast(pw, jnp.bfloat16)
                        * plsc.bitcast(tw, jnp.bfloat16)
                        * plsc.bitcast(rw, jnp.bfloat16))
                ua, ub = plsc.unpack(prod, format=plsc.PackFormat.INTERLEAVED)
                acc = acc + ua + ub
            for sh in (1, 2, 4, 8):
                acc = acc + jnp.take(acc, iota ^ sh)
            s = acc[0]
            if k % GROUP == 0:
                cur = s
            else:
                lsum = lsum + jnp.maximum(s - cur + 1.0, 0.0)
        return lsum

    def prime(g, acc):
        issue_group(0, g, p0, t0)
        return acc

    lax.fori_loop(0, GPC, prime, jnp.int32(0))

    def half(cc, buf_pair, next_pair, lsum):
        drain()
        pb, tb = buf_pair
        npb, ntb = next_pair

        @pl.when(cc + 1 < NCHUNK)
        def _():
            def i_body(g, acc):
                issue_group(cc + 1, g, npb, ntb)
                return acc
            lax.fori_loop(0, GPC, i_body, jnp.int32(0))

        def g_body(g, lsum):
            return compute_group(cc, g, pb, tb, lsum)

        return lax.fori_loop(0, GPC, g_body, lsum)

    def pair_body(p2, lsum):
        cc = pl.multiple_of(p2 * 2, 2)
        lsum = half(cc, (p0, t0), (p1, t1), lsum)
        lsum = half(cc + 1, (p1, t1), (p0, t0), lsum)
        return lsum

    lsum = lax.fori_loop(0, NCHUNK // 2, pair_body, jnp.float32(0.0))

    partial_v[...] = jnp.where(iota == 0, lsum * (GROUP / BATCH), 0.0)
    pltpu.sync_copy(partial_v, out_hbm.at[wid])


_mesh = plsc.VectorSubcoreMesh(core_axis_name="c", subcore_axis_name="s")

_sc_call = pl.kernel(
    _body,
    out_type=jax.ShapeDtypeStruct((NW, L), jnp.float32),
    mesh=_mesh,
    scratch_types=[
        pltpu.VMEM((BPW,), jnp.int32),
        pltpu.VMEM((BPW,), jnp.int32),
        pltpu.VMEM((BPW,), jnp.int32),
        pltpu.VMEM((NUM_RELATION_TYPES * WROW,), jnp.float32),
        pltpu.VMEM((CROWS,), jnp.float32),
        pltpu.VMEM((CROWS,), jnp.float32),
        pltpu.VMEM((CROWS,), jnp.float32),
        pltpu.VMEM((CROWS,), jnp.float32),
        pltpu.VMEM((L,), jnp.float32),
        pltpu.SemaphoreType.DMA,
    ],
    compiler_params=pltpu.CompilerParams(needs_layout_passes=False),
)


def _pack_rows(table):
    t16 = table.astype(jnp.bfloat16)
    t16 = jnp.pad(t16, ((0, 0), (0, DPAD - EMBED_SIZE)))
    tw = jax.lax.bitcast_convert_type(
        t16.reshape(-1, WROW, 2), jnp.float32)
    return tw.reshape(-1)


@jax.jit
def kernel(problems, rels, targets, labels, prob_embed, ord_embed, rel_embed):
    del labels
    out = _sc_call(problems.astype(jnp.int32), rels.astype(jnp.int32),
                   targets.astype(jnp.int32), _pack_rows(prob_embed),
                   _pack_rows(ord_embed), _pack_rows(rel_embed))
    return jnp.sum(out)


# speedup vs baseline: 5.4588x; 5.4588x over previous
import jax
import jax.numpy as jnp
from jax import lax
from jax.experimental import pallas as pl
from jax.experimental.pallas import tpu as pltpu
from jax.experimental.pallas import tpu_sc as plsc

NUM_RELATION_TYPES = 3
EMBED_SIZE = 300
BATCH = 16384
GROUP = 4

NC = 2
NS = 16
NW = NC * NS
L = 16
BPW = BATCH // NW
CHUNK = 64
NCHUNK = BPW // CHUNK
GPC = CHUNK // L
NFULL = EMBED_SIZE // L


def _body(problems_hbm, rels_hbm, targets_hbm, prob_hbm, ord_hbm, rel_hbm,
          out_hbm, pidx_v, tidx_v, ridx_v, rel_v, p0, t0, p1, t1,
          partial_v, sem):
    wid = lax.axis_index("s") * NC + lax.axis_index("c")
    base = wid * BPW

    pltpu.sync_copy(problems_hbm.at[pl.ds(base, BPW)], pidx_v)
    pltpu.sync_copy(targets_hbm.at[pl.ds(base, BPW)], tidx_v)
    pltpu.sync_copy(rels_hbm.at[pl.ds(base, BPW)], ridx_v)
    pltpu.sync_copy(rel_hbm, rel_v)

    iota = lax.iota(jnp.int32, L)
    tail_mask = iota >= (L - (EMBED_SIZE - NFULL * L))
    tail_off = EMBED_SIZE - L

    def issue_group(c, g, p_buf, t_buf):
        goff = pl.multiple_of(g * L, L)
        off = pl.multiple_of(c * CHUNK, CHUNK) + goff
        pidx = pidx_v[pl.ds(off, L)]
        tidx = tidx_v[pl.ds(off, L)]
        for k in range(L):
            pltpu.async_copy(prob_hbm.at[pidx[k]], p_buf.at[goff + k], sem)
            pltpu.async_copy(ord_hbm.at[tidx[k]], t_buf.at[goff + k], sem)

    def drain():
        pltpu.make_async_copy(prob_hbm.at[pl.ds(0, CHUNK)], p0, sem).wait()
        pltpu.make_async_copy(ord_hbm.at[pl.ds(0, CHUNK)], t0, sem).wait()

    def compute_group(c, g, p_buf, t_buf, lsum):
        goff = pl.multiple_of(g * L, L)
        coff = pl.multiple_of(c * CHUNK, CHUNK)
        rid_vec = ridx_v[pl.ds(coff + goff, L)]
        cur = jnp.float32(0.0)
        for k in range(L):
            i = goff + k
            rid = rid_vec[k]
            acc = jnp.zeros((L,), jnp.float32)
            for j in range(NFULL):
                pj = p_buf[i, pl.ds(j * L, L)]
                tj = t_buf[i, pl.ds(j * L, L)]
                rj = rel_v[rid, pl.ds(j * L, L)]
                acc = acc + pj * tj * rj
            pj = p_buf[i, pl.ds(tail_off, L)]
            tj = t_buf[i, pl.ds(tail_off, L)]
            rj = rel_v[rid, pl.ds(tail_off, L)]
            acc = acc + jnp.where(tail_mask, pj * tj * rj, 0.0)
            for sh in (1, 2, 4, 8):
                acc = acc + jnp.take(acc, iota ^ sh)
            s = acc[0]
            if k % GROUP == 0:
                cur = s
            else:
                lsum = lsum + jnp.maximum(s - cur + 1.0, 0.0)
        return lsum

    def prime(g, acc):
        issue_group(0, g, p0, t0)
        return acc

    lax.fori_loop(0, GPC, prime, jnp.int32(0))

    def half(cc, buf_pair, next_pair, lsum):
        drain()
        pb, tb = buf_pair
        npb, ntb = next_pair

        @pl.when(cc + 1 < NCHUNK)
        def _():
            def i_body(g, acc):
                issue_group(cc + 1, g, npb, ntb)
                return acc
            lax.fori_loop(0, GPC, i_body, jnp.int32(0))

        def g_body(g, lsum):
            return compute_group(cc, g, pb, tb, lsum)

        return lax.fori_loop(0, GPC, g_body, lsum)

    def pair_body(p2, lsum):
        cc = pl.multiple_of(p2 * 2, 2)
        lsum = half(cc, (p0, t0), (p1, t1), lsum)
        lsum = half(cc + 1, (p1, t1), (p0, t0), lsum)
        return lsum

    lsum = lax.fori_loop(0, NCHUNK // 2, pair_body, jnp.float32(0.0))

    partial_v[...] = jnp.where(iota == 0, lsum * (GROUP / BATCH), 0.0)
    pltpu.sync_copy(partial_v, out_hbm.at[wid])


_mesh = plsc.VectorSubcoreMesh(core_axis_name="c", subcore_axis_name="s")

_sc_call = pl.kernel(
    _body,
    out_type=jax.ShapeDtypeStruct((NW, L), jnp.float32),
    mesh=_mesh,
    scratch_types=[
        pltpu.VMEM((BPW,), jnp.int32),
        pltpu.VMEM((BPW,), jnp.int32),
        pltpu.VMEM((BPW,), jnp.int32),
        pltpu.VMEM((NUM_RELATION_TYPES, EMBED_SIZE), jnp.float32),
        pltpu.VMEM((CHUNK, EMBED_SIZE), jnp.float32),
        pltpu.VMEM((CHUNK, EMBED_SIZE), jnp.float32),
        pltpu.VMEM((CHUNK, EMBED_SIZE), jnp.float32),
        pltpu.VMEM((CHUNK, EMBED_SIZE), jnp.float32),
        pltpu.VMEM((L,), jnp.float32),
        pltpu.SemaphoreType.DMA,
    ],
    compiler_params=pltpu.CompilerParams(needs_layout_passes=False),
)


@jax.jit
def kernel(problems, rels, targets, labels, prob_embed, ord_embed, rel_embed):
    del labels
    out = _sc_call(problems.astype(jnp.int32), rels.astype(jnp.int32),
                   targets.astype(jnp.int32), prob_embed, ord_embed,
                   rel_embed)
    return jnp.sum(out)
